# baseline (speedup 1.0000x reference)
import functools

import jax
import jax.numpy as jnp
from jax import lax
from jax.experimental import pallas as pl
from jax.experimental.pallas import tpu as pltpu
from jax.experimental.pallas import tpu_sc as plsc

NC = 2
NS = 16
NW = NC * NS
L = 16
D = 256
P = 25
PPAD = 32


def _body(b_per_w, n_chunk, ch,
          patterns_hbm, scale_hbm, shift_hbm, idx_hbm, out_hbm,
          pat_v, sc_v, sh_v, tab_v, idx_v, st0, st1, p0, p1):
    w = lax.axis_index("s") * NC + lax.axis_index("c")
    base = w * b_per_w
    pltpu.sync_copy(patterns_hbm, pat_v)
    pltpu.sync_copy(scale_hbm, sc_v)
    pltpu.sync_copy(shift_hbm, sh_v)
    pltpu.sync_copy(idx_hbm.at[pl.ds(base, b_per_w)], idx_v)

    def build_row(p, _):
        pv = jnp.full((L,), p, jnp.int32)
        scale = plsc.load_gather(sc_v, [pv])
        shf = plsc.load_gather(sh_v, [pv])
        s = (shf * 10.0).astype(jnp.int32)
        for c in range(D // L):
            col = lax.iota(jnp.int32, L) + (c * L)
            src = lax.rem(col - s, D)
            src = src + jnp.where(src < 0, D, 0)
            vals = plsc.load_gather(pat_v, [pv, src])
            tab_v[pl.ds(p * D + c * L, L)] = vals * scale
        return 0

    lax.fori_loop(0, P, build_row, 0)

    sts = (st0, st1)
    ps = (p0, p1)

    def construct(c, st):
        def group(g, _):
            ivec = idx_v[pl.ds(c * ch + g * L, L)]
            for k in range(L):
                rb = ivec[k] * D
                ob = g * L + k
                for q in range(D // L):
                    st[ob, pl.ds(q * L, L)] = tab_v[pl.ds(rb + q * L, L)]
            return 0

        lax.fori_loop(0, ch // L, group, 0)

    def put(c, b):
        return pltpu.make_async_copy(
            sts[b], out_hbm.at[pl.ds(base + c * ch, ch)], ps[b])

    construct(0, st0)
    put(0, 0).start()
    construct(1, st1)
    put(1, 1).start()

    def body(i, _):
        c = 2 * i + 2
        put(c - 2, 0).wait()
        construct(c, st0)
        put(c, 0).start()
        put(c - 1, 1).wait()
        construct(c + 1, st1)
        put(c + 1, 1).start()
        return 0

    lax.fori_loop(0, (n_chunk - 2) // 2, body, 0)
    put(n_chunk - 2, 0).wait()
    put(n_chunk - 1, 1).wait()


def kernel(indices, patterns, amplitude_scale, frequency_shift):
    bsz, seq = indices.shape
    n = bsz * seq
    b_per_w = n // NW
    ch = 128
    n_chunk = b_per_w // ch

    mesh = plsc.VectorSubcoreMesh(
        core_axis_name="c", subcore_axis_name="s",
        num_cores=NC, num_subcores=NS)

    scale_p = jnp.zeros((PPAD,), jnp.float32).at[:P].set(amplitude_scale)
    shift_p = jnp.zeros((PPAD,), jnp.float32).at[:P].set(frequency_shift)

    run = pl.kernel(
        functools.partial(_body, b_per_w, n_chunk, ch),
        out_type=jax.ShapeDtypeStruct((n, D), jnp.float32),
        mesh=mesh,
        compiler_params=pltpu.CompilerParams(needs_layout_passes=False),
        scratch_types=[
            pltpu.VMEM((P, D), jnp.float32),
            pltpu.VMEM((PPAD,), jnp.float32),
            pltpu.VMEM((PPAD,), jnp.float32),
            pltpu.VMEM((P * D,), jnp.float32),
            pltpu.VMEM((b_per_w,), jnp.int32),
            pltpu.VMEM((ch, D), jnp.float32),
            pltpu.VMEM((ch, D), jnp.float32),
            pltpu.SemaphoreType.DMA,
            pltpu.SemaphoreType.DMA,
        ],
    )
    out = run(patterns, scale_p, shift_p, indices.reshape(n))
    return out.reshape(bsz, seq, D)
